# Initial kernel scaffold; baseline (speedup 1.0000x reference)
import jax
import jax.numpy as jnp
from jax.experimental import pallas as pl


def kernel(user_id, sex, age, occupation, seq_item, target_item, user_table, sex_table, age_table, occupation_table, movie_table):
    raise NotImplementedError("write your pallas kernel here")


# speedup vs baseline: 2.2498x; 2.2498x over previous
import functools

import jax
import jax.numpy as jnp
from jax import lax
from jax.experimental import pallas as pl
from jax.experimental.pallas import tpu as pltpu
from jax.experimental.pallas import tpu_sc as plsc

B = 4096
L = 200
D_USER = 64
D_SMALL = 16
D_MOVIE = 64
D_FEAT = D_USER + 3 * D_SMALL

_INFO = plsc.get_sparse_core_info()
NC = _INFO.num_cores
NS = _INFO.num_subcores
NW = NC * NS

BPW = B // NW
SEQ_TOTAL = B * L
SEQ_PW = SEQ_TOTAL // NW
IDXW = 128
CH = 4
CHUNK_ROWS = CH * IDXW
N_CHUNKS = SEQ_PW // CHUNK_ROWS
IDX_ROWS_PW = SEQ_PW // IDXW


def _body(user_id, sex, age, occupation, seq_idx2d, target_item,
          user_table, sex_table, age_table, occupation_table, movie_table,
          feat_out, seq_out, tgt_out,
          uidx, sidx, aidx, oidx, tidx,
          urows, srows, arows, orows, trows, feat_v,
          idx_v, rows_v, sem):
    wid = lax.axis_index("s") * NC + lax.axis_index("c")
    base = wid * BPW

    pltpu.sync_copy(user_id.at[pl.ds(base, BPW)], uidx)
    pltpu.sync_copy(sex.at[pl.ds(base, BPW)], sidx)
    pltpu.sync_copy(age.at[pl.ds(base, BPW)], aidx)
    pltpu.sync_copy(occupation.at[pl.ds(base, BPW)], oidx)
    pltpu.sync_copy(target_item.at[pl.ds(base, BPW)], tidx)

    cps = [
        pltpu.async_copy(user_table.at[uidx], urows, sem),
        pltpu.async_copy(sex_table.at[sidx], srows, sem),
        pltpu.async_copy(age_table.at[aidx], arows, sem),
        pltpu.async_copy(occupation_table.at[oidx], orows, sem),
        pltpu.async_copy(movie_table.at[tidx], trows, sem),
    ]
    for cp in cps:
        cp.wait()

    def assemble(r, _):
        for c in range(D_USER // 16):
            feat_v[r, pl.ds(16 * c, 16)] = urows[r, pl.ds(16 * c, 16)]
        feat_v[r, pl.ds(D_USER, 16)] = srows[r, :]
        feat_v[r, pl.ds(D_USER + D_SMALL, 16)] = arows[r, :]
        feat_v[r, pl.ds(D_USER + 2 * D_SMALL, 16)] = orows[r, :]
        return _

    lax.fori_loop(0, BPW, assemble, None)
    pltpu.sync_copy(feat_v, feat_out.at[pl.ds(base, BPW)])
    pltpu.sync_copy(trows, tgt_out.at[pl.ds(base, BPW)])

    idx_row0 = wid * IDX_ROWS_PW
    out_row0 = wid * SEQ_PW

    def chunk(g, _):
        pltpu.sync_copy(seq_idx2d.at[pl.ds(idx_row0 + g * CH, CH)], idx_v)
        gathers = []
        for j in range(CH):
            gathers.append(pltpu.async_copy(
                movie_table.at[idx_v.at[j]],
                rows_v.at[pl.ds(j * IDXW, IDXW)], sem))
        for cp in gathers:
            cp.wait()
        pltpu.sync_copy(rows_v, seq_out.at[pl.ds(out_row0 + g * CHUNK_ROWS, CHUNK_ROWS)])
        return _

    lax.fori_loop(0, N_CHUNKS, chunk, None)


@functools.partial(jax.jit, static_argnums=())
def _run(user_id, sex, age, occupation, seq_idx2d, target_item,
         user_table, sex_table, age_table, occupation_table, movie_table):
    mesh = plsc.VectorSubcoreMesh(core_axis_name="c", subcore_axis_name="s")
    k = functools.partial(
        pl.kernel,
        mesh=mesh,
        compiler_params=pltpu.CompilerParams(use_tc_tiling_on_sc=False),
        out_type=[
            jax.ShapeDtypeStruct((B, D_FEAT), jnp.float32),
            jax.ShapeDtypeStruct((SEQ_TOTAL, D_MOVIE), jnp.float32),
            jax.ShapeDtypeStruct((B, D_MOVIE), jnp.float32),
        ],
        scratch_types=[
            pltpu.VMEM((BPW,), jnp.int32),
            pltpu.VMEM((BPW,), jnp.int32),
            pltpu.VMEM((BPW,), jnp.int32),
            pltpu.VMEM((BPW,), jnp.int32),
            pltpu.VMEM((BPW,), jnp.int32),
            pltpu.VMEM((BPW, D_USER), jnp.float32),
            pltpu.VMEM((BPW, D_SMALL), jnp.float32),
            pltpu.VMEM((BPW, D_SMALL), jnp.float32),
            pltpu.VMEM((BPW, D_SMALL), jnp.float32),
            pltpu.VMEM((BPW, D_MOVIE), jnp.float32),
            pltpu.VMEM((BPW, D_FEAT), jnp.float32),
            pltpu.VMEM((CH, IDXW), jnp.int32),
            pltpu.VMEM((CHUNK_ROWS, D_MOVIE), jnp.float32),
            pltpu.SemaphoreType.DMA,
        ],
    )(_body)
    return k(user_id, sex, age, occupation, seq_idx2d, target_item,
             user_table, sex_table, age_table, occupation_table, movie_table)


def kernel(user_id, sex, age, occupation, seq_item, target_item,
           user_table, sex_table, age_table, occupation_table, movie_table):
    seq_idx2d = seq_item.reshape(SEQ_TOTAL // IDXW, IDXW)
    feat, seq_flat, tgt = _run(
        user_id.astype(jnp.int32), sex.astype(jnp.int32), age.astype(jnp.int32),
        occupation.astype(jnp.int32), seq_idx2d.astype(jnp.int32),
        target_item.astype(jnp.int32),
        user_table, sex_table, age_table, occupation_table, movie_table)
    return (feat, seq_flat.reshape(B, L, D_MOVIE), tgt)


# speedup vs baseline: 2.3539x; 1.0463x over previous
import functools

import jax
import jax.numpy as jnp
from jax import lax
from jax.experimental import pallas as pl
from jax.experimental.pallas import tpu as pltpu
from jax.experimental.pallas import tpu_sc as plsc

B = 4096
L = 200
D_USER = 64
D_SMALL = 16
D_MOVIE = 64
D_FEAT = D_USER + 3 * D_SMALL

_INFO = plsc.get_sparse_core_info()
NC = _INFO.num_cores
NS = _INFO.num_subcores
NW = NC * NS

BPW = B // NW
SEQ_TOTAL = B * L
SEQ_PW = SEQ_TOTAL // NW
IDXW = 128
NB = 8
N_BLOCKS = SEQ_PW // (NB * IDXW)
IDX_ROWS_PW = SEQ_PW // IDXW


def _body(user_id, sex, age, occupation, seq_idx2d, target_item,
          user_table, sex_table, age_table, occupation_table, movie_table,
          feat_out, seq_out, tgt_out,
          uidx, sidx, aidx, oidx, tidx,
          urows, srows, arows, orows, trows, feat_v,
          idx_v, rows_v, sem, gsem, wsem, isem):
    wid = lax.axis_index("s") * NC + lax.axis_index("c")
    base = wid * BPW

    pltpu.sync_copy(user_id.at[pl.ds(base, BPW)], uidx)
    pltpu.sync_copy(sex.at[pl.ds(base, BPW)], sidx)
    pltpu.sync_copy(age.at[pl.ds(base, BPW)], aidx)
    pltpu.sync_copy(occupation.at[pl.ds(base, BPW)], oidx)
    pltpu.sync_copy(target_item.at[pl.ds(base, BPW)], tidx)

    cps = [
        pltpu.async_copy(user_table.at[uidx], urows, sem),
        pltpu.async_copy(sex_table.at[sidx], srows, sem),
        pltpu.async_copy(age_table.at[aidx], arows, sem),
        pltpu.async_copy(occupation_table.at[oidx], orows, sem),
        pltpu.async_copy(movie_table.at[tidx], trows, sem),
    ]
    for cp in cps:
        cp.wait()

    def assemble(r, _):
        for c in range(D_USER // 16):
            feat_v[r, pl.ds(16 * c, 16)] = urows[r, pl.ds(16 * c, 16)]
        feat_v[r, pl.ds(D_USER, 16)] = srows[r, :]
        feat_v[r, pl.ds(D_USER + D_SMALL, 16)] = arows[r, :]
        feat_v[r, pl.ds(D_USER + 2 * D_SMALL, 16)] = orows[r, :]
        return _

    lax.fori_loop(0, BPW, assemble, None)
    pltpu.sync_copy(feat_v, feat_out.at[pl.ds(base, BPW)])
    pltpu.sync_copy(trows, tgt_out.at[pl.ds(base, BPW)])

    idx_row0 = wid * IDX_ROWS_PW
    out_row0 = wid * SEQ_PW

    def out_slice(blk, b):
        return seq_out.at[pl.ds(out_row0 + (blk * NB + b) * IDXW, IDXW)]

    def idx_load(blk, slot):
        return pltpu.async_copy(
            seq_idx2d.at[pl.ds(idx_row0 + blk * NB, NB)], idx_v.at[slot], isem)

    def gather(slot, b):
        return pltpu.async_copy(
            movie_table.at[idx_v.at[slot, b]], rows_v.at[b], gsem)

    idx_load(0, 0).wait()
    idx_load(1, 1)
    for b in range(NB):
        gather(0, b)

    def block(i, _):
        cur = i % 2
        nxt = (i + 1) % 2
        for b in range(NB):
            pltpu.make_async_copy(movie_table.at[idx_v.at[cur, b]],
                                  rows_v.at[b], gsem).wait()
            pltpu.async_copy(rows_v.at[b], out_slice(i, b), wsem)

        @pl.when(i < N_BLOCKS - 1)
        def _refill():
            pltpu.make_async_copy(
                seq_idx2d.at[pl.ds(idx_row0 + (i + 1) * NB, NB)],
                idx_v.at[nxt], isem).wait()

            @pl.when(i < N_BLOCKS - 2)
            def _next_idx():
                idx_load(i + 2, cur)

            for b in range(NB):
                pltpu.make_async_copy(rows_v.at[b], out_slice(i, b), wsem).wait()
                gather(nxt, b)

        @pl.when(i == N_BLOCKS - 1)
        def _drain():
            for b in range(NB):
                pltpu.make_async_copy(rows_v.at[b], out_slice(i, b), wsem).wait()

        return _

    lax.fori_loop(0, N_BLOCKS, block, None)


@functools.partial(jax.jit, static_argnums=())
def _run(user_id, sex, age, occupation, seq_idx2d, target_item,
         user_table, sex_table, age_table, occupation_table, movie_table):
    mesh = plsc.VectorSubcoreMesh(core_axis_name="c", subcore_axis_name="s")
    k = functools.partial(
        pl.kernel,
        mesh=mesh,
        compiler_params=pltpu.CompilerParams(use_tc_tiling_on_sc=False),
        out_type=[
            jax.ShapeDtypeStruct((B, D_FEAT), jnp.float32),
            jax.ShapeDtypeStruct((SEQ_TOTAL, D_MOVIE), jnp.float32),
            jax.ShapeDtypeStruct((B, D_MOVIE), jnp.float32),
        ],
        scratch_types=[
            pltpu.VMEM((BPW,), jnp.int32),
            pltpu.VMEM((BPW,), jnp.int32),
            pltpu.VMEM((BPW,), jnp.int32),
            pltpu.VMEM((BPW,), jnp.int32),
            pltpu.VMEM((BPW,), jnp.int32),
            pltpu.VMEM((BPW, D_USER), jnp.float32),
            pltpu.VMEM((BPW, D_SMALL), jnp.float32),
            pltpu.VMEM((BPW, D_SMALL), jnp.float32),
            pltpu.VMEM((BPW, D_SMALL), jnp.float32),
            pltpu.VMEM((BPW, D_MOVIE), jnp.float32),
            pltpu.VMEM((BPW, D_FEAT), jnp.float32),
            pltpu.VMEM((2, NB, IDXW), jnp.int32),
            pltpu.VMEM((NB, IDXW, D_MOVIE), jnp.float32),
            pltpu.SemaphoreType.DMA,
            pltpu.SemaphoreType.DMA,
            pltpu.SemaphoreType.DMA,
            pltpu.SemaphoreType.DMA,
        ],
    )(_body)
    return k(user_id, sex, age, occupation, seq_idx2d, target_item,
             user_table, sex_table, age_table, occupation_table, movie_table)


def kernel(user_id, sex, age, occupation, seq_item, target_item,
           user_table, sex_table, age_table, occupation_table, movie_table):
    seq_idx2d = seq_item.reshape(SEQ_TOTAL // IDXW, IDXW)
    feat, seq_flat, tgt = _run(
        user_id.astype(jnp.int32), sex.astype(jnp.int32), age.astype(jnp.int32),
        occupation.astype(jnp.int32), seq_idx2d.astype(jnp.int32),
        target_item.astype(jnp.int32),
        user_table, sex_table, age_table, occupation_table, movie_table)
    return (feat, seq_flat.reshape(B, L, D_MOVIE), tgt)
